# baseline (speedup 1.0000x reference)
import functools

import jax
import jax.numpy as jnp
from jax import lax
from jax.experimental import pallas as pl
from jax.experimental.pallas import tpu as pltpu
from jax.experimental.pallas import tpu_sc as plsc

_E = 64
_K = 8
_TILE = 1024
_L = 16
_NW = 32
_HALF = 512


def _matmul_body(x_ref, w_ref, lt_ref):
    x = x_ref[0]
    w = w_ref[...]
    logits = lax.dot_general(
        x, w, (((1,), (1,)), ((), ())),
        preferred_element_type=jnp.float32)
    lt_ref[...] = logits.T


def _tc_logits(input, W):
    b, s, d = input.shape
    n = b * s
    nj = s // _TILE
    return pl.pallas_call(
        _matmul_body,
        grid=(b, nj),
        in_specs=[
            pl.BlockSpec((1, _TILE, d), lambda i, j: (i, j, 0)),
            pl.BlockSpec((_E, d), lambda i, j: (0, 0)),
        ],
        out_specs=pl.BlockSpec((_E, _TILE), lambda i, j: (0, i * nj + j)),
        out_shape=jax.ShapeDtypeStruct((_E, n), jnp.float32),
        compiler_params=pltpu.CompilerParams(
            dimension_semantics=("arbitrary", "arbitrary"),
        ),
    )(input, W)


def _make_route(n):
    sub = n // _NW
    mesh = plsc.VectorSubcoreMesh(core_axis_name="c", subcore_axis_name="s")

    @functools.partial(
        pl.kernel, mesh=mesh,
        out_type=[
            jax.ShapeDtypeStruct((n * _E,), jnp.float32),
            jax.ShapeDtypeStruct((n * _K,), jnp.int32),
        ],
        scratch_types=[
            pltpu.VMEM((_E, _HALF), jnp.float32),
            pltpu.VMEM((_HALF * _E,), jnp.float32),
            pltpu.VMEM((_HALF * _K,), jnp.int32),
        ],
        compiler_params=pltpu.CompilerParams(needs_layout_passes=False),
    )
    def route(lt_hbm, zz_hbm, w_hbm, idx_hbm, lt_v, w_v, idx_v):
        wid = lax.axis_index("s") * 2 + lax.axis_index("c")
        tok0 = wid * sub
        lane = lax.broadcasted_iota(jnp.int32, (_L,), 0)
        neginf = jnp.full((_L,), -jnp.inf, jnp.float32)

        def half_body(h, carry):
            base = tok0 + h * _HALF
            pltpu.sync_copy(lt_hbm.at[:, pl.ds(base, _HALF)], lt_v)
            pltpu.sync_copy(zz_hbm, w_v)

            def group_body(g, c):
                work = [lt_v[e, pl.ds(g * _L, _L)] for e in range(_E)]
                ms, ams = [], []
                for j in range(_K):
                    cur_v = work
                    cur_i = [jnp.full((_L,), e, jnp.int32) for e in range(_E)]
                    while len(cur_v) > 1:
                        nv, ni = [], []
                        for a in range(0, len(cur_v), 2):
                            take = cur_v[a + 1] > cur_v[a]
                            nv.append(jnp.maximum(cur_v[a], cur_v[a + 1]))
                            ni.append(jnp.where(take, cur_i[a + 1], cur_i[a]))
                        cur_v, cur_i = nv, ni
                    m, am = cur_v[0], cur_i[0]
                    ms.append(m)
                    ams.append(am)
                    if j + 1 < _K:
                        work = [jnp.where(am == e, neginf, work[e])
                                for e in range(_E)]
                exps = [jnp.exp(mj - ms[0]) for mj in ms]
                den = exps[0]
                for t in exps[1:]:
                    den = den + t
                tok = g * _L + lane
                wbase = tok * _E
                ibase = tok * _K
                for j in range(_K):
                    plsc.store_scatter(w_v, [wbase + ams[j]], exps[j] / den)
                    plsc.store_scatter(idx_v, [ibase + j], ams[j])
                return c

            lax.fori_loop(0, _HALF // _L, group_body, 0)
            pltpu.sync_copy(w_v, w_hbm.at[pl.ds(base * _E, _HALF * _E)])
            pltpu.sync_copy(idx_v, idx_hbm.at[pl.ds(base * _K, _HALF * _K)])
            return carry

        lax.fori_loop(0, sub // _HALF, half_body, 0)

    return route


def kernel(input, W):
    b, s, d = input.shape
    n = b * s
    lt = _tc_logits(input, W)
    zz = jnp.zeros((_HALF * _E,), jnp.float32)
    w_flat, idx_flat = _make_route(n)(lt, zz)
    return (w_flat.reshape(b, s, _E), idx_flat.reshape(b, s, _K))
